# baseline (speedup 1.0000x reference)
import functools

import jax
import jax.numpy as jnp
from jax import lax
from jax.experimental import pallas as pl
from jax.experimental.pallas import tpu as pltpu
from jax.experimental.pallas import tpu_sc as plsc

N = 10000
D = 128
E = 320000
NC = 2
NS = 16
NW = NC * NS
CH = 128
NCHUNK = ((-(-E // (NW * CH)) + 7) // 8) * 8
EPT = NCHUNK * CH
EPAD = EPT * NW
GB = 8
NG = NCHUNK // GB
K0 = 128
K1 = 2 * NCHUNK - K0
NOUT = 10240
RPT = NOUT // NS
BR = 512

_mesh = plsc.VectorSubcoreMesh(core_axis_name="c", subcore_axis_name="s")
_sc_params = pltpu.CompilerParams(needs_layout_passes=False)


@functools.partial(
    pl.kernel,
    mesh=_mesh,
    out_type=jax.ShapeDtypeStruct((NC, NOUT, D), jnp.float32),
    scratch_types=[
        pltpu.VMEM((GB, CH), jnp.int32),
        pltpu.VMEM((GB, CH), jnp.int32),
        pltpu.VMEM((GB, CH), jnp.int32),
        pltpu.VMEM((GB, CH), jnp.int32),
        pltpu.VMEM((2 * CH, D), jnp.float32),
        pltpu.VMEM_SHARED((NOUT, D), jnp.float32),
        pltpu.SemaphoreType.DMA,
        pltpu.SemaphoreType.DMA,
        pltpu.SemaphoreType.DMA,
        pltpu.SemaphoreType.DMA,
        pltpu.SemaphoreType.DMA,
    ],
)
def _sc_agg(x, srcp, dstp, zrows, out,
            srcA, dstA, srcB, dstB, rows2, agg_sh, g0, g1, s0, s1, isem):
  c = lax.axis_index("c")
  s = lax.axis_index("s")
  r0 = s * RPT

  gsem = (g0, g1)
  ssem = (s0, s1)
  bufs = ((srcA, dstA), (srcB, dstB))

  def rslot(u):
    return rows2.at[pl.ds(u * CH, CH)]

  def gather_start(sbuf, j, u):
    pltpu.async_copy(x.at[sbuf.at[j]], rslot(u), gsem[u])

  def gather_wait(sbuf, j, u):
    pltpu.make_async_copy(x.at[sbuf.at[j]], rslot(u), gsem[u]).wait()

  def scatter_start(dbuf, j, u):
    pltpu.async_copy(rslot(u), agg_sh.at[dbuf.at[j]], ssem[u], add=True)

  def scatter_wait(dbuf, j, u):
    pltpu.make_async_copy(rslot(u), agg_sh.at[dbuf.at[j]], ssem[u]).wait()

  def pipe(base, ngk):
    def idx_start(m, par):
      sbuf, dbuf = bufs[par]
      pltpu.async_copy(srcp.at[pl.ds(base + m * GB, GB)], sbuf, isem)
      pltpu.async_copy(dstp.at[pl.ds(base + m * GB, GB)], dbuf, isem)

    def idx_wait(m, par):
      sbuf, dbuf = bufs[par]
      pltpu.make_async_copy(srcp.at[pl.ds(base + m * GB, GB)], sbuf, isem).wait()
      pltpu.make_async_copy(dstp.at[pl.ds(base + m * GB, GB)], dbuf, isem).wait()

    def emit_group(m, par, first=False, last=False):
      sbuf, dbuf = bufs[par]
      psbuf, pdbuf = bufs[1 - par]
      for u in range(GB):
        tp = u % 2
        if u == 0:
          if not first:
            scatter_wait(pdbuf, GB - 1, 1)
          if not last:
            idx_start(m + 1, 1 - par)
        else:
          scatter_wait(dbuf, u - 1, (u - 1) % 2)
        if u < GB - 1:
          gather_start(sbuf, u + 1, (u + 1) % 2)
        elif not last:
          idx_wait(m + 1, 1 - par)
          gather_start(psbuf, 0, 0)
        gather_wait(sbuf, u, tp)
        scatter_start(dbuf, u, tp)

    pltpu.sync_copy(srcp.at[pl.ds(base, GB)], srcA)
    pltpu.sync_copy(dstp.at[pl.ds(base, GB)], dstA)
    gather_start(srcA, 0, 0)
    emit_group(0, 0, first=True)
    emit_group(1, 1)

    def pair(k, carry):
      emit_group(2 * k, 0)
      emit_group(2 * k + 1, 1)
      return carry

    lax.fori_loop(1, ngk // 2 - 1, pair, 0)

    emit_group(ngk - 2, 0)
    emit_group(ngk - 1, 1, last=True)
    scatter_wait(bufs[1][1], GB - 1, 1)

  pltpu.sync_copy(zrows, agg_sh.at[pl.ds(r0, RPT)])
  plsc.subcore_barrier()
  lax.cond(
      c == 0,
      lambda: pipe(s * K0, K0 // GB),
      lambda: pipe(NS * K0 + s * K1, K1 // GB),
  )
  plsc.subcore_barrier()
  pltpu.sync_copy(agg_sh.at[pl.ds(r0, RPT)], out.at[c, pl.ds(r0, RPT)])


@functools.partial(
    pl.kernel,
    mesh=_mesh,
    out_type=jax.ShapeDtypeStruct((NW, NOUT), jnp.float32),
    scratch_types=[
        pltpu.VMEM((NCHUNK, CH), jnp.int32),
        pltpu.VMEM((NOUT,), jnp.float32),
    ],
    compiler_params=_sc_params,
)
def _sc_cnt(dstp, zcnt, out, dst_all, cnt_v):
  c = lax.axis_index("c")
  s = lax.axis_index("s")
  w = s * NC + c
  pltpu.sync_copy(zcnt, cnt_v)
  pltpu.sync_copy(dstp.at[pl.ds(w * NCHUNK, NCHUNK)], dst_all)
  ones16 = jnp.ones((16,), jnp.float32)

  def chunk(j, carry):
    for k in range(CH // 16):
      idx = dst_all[j, pl.ds(k * 16, 16)]
      plsc.addupdate_scatter(cnt_v, [idx], ones16)
    return carry

  lax.fori_loop(0, NCHUNK, chunk, 0)
  pltpu.sync_copy(cnt_v, out.at[w])


def _tc_body(agg_ref, cnt_ref, x_ref, wl_ref, wr_ref, b_ref, out_ref):
  a = agg_ref[0] + agg_ref[1]
  cnt = jnp.sum(cnt_ref[...], axis=1, keepdims=True)
  mean = a / jnp.maximum(cnt, 1.0)
  acc = jnp.dot(mean, wl_ref[...], preferred_element_type=jnp.float32,
                precision=lax.Precision.HIGHEST)
  acc = acc + jnp.dot(x_ref[...], wr_ref[...], preferred_element_type=jnp.float32,
                      precision=lax.Precision.HIGHEST)
  out_ref[...] = jnp.tanh(acc + b_ref[...])


def _tc_layer(agg, cnt_t, x, wl, wr, b):
  return pl.pallas_call(
      _tc_body,
      grid=(NOUT // BR,),
      in_specs=[
          pl.BlockSpec((NC, BR, D), lambda i: (0, i, 0)),
          pl.BlockSpec((BR, NW), lambda i: (i, 0)),
          pl.BlockSpec((BR, D), lambda i: (i, 0)),
          pl.BlockSpec((D, D), lambda i: (0, 0)),
          pl.BlockSpec((D, D), lambda i: (0, 0)),
          pl.BlockSpec((1, D), lambda i: (0, 0)),
      ],
      out_specs=pl.BlockSpec((BR, D), lambda i: (i, 0)),
      out_shape=jax.ShapeDtypeStruct((N, D), jnp.float32),
  )(agg, cnt_t, x, wl, wr, b)


def _copy_body(x_ref, o_ref):
  o_ref[...] = x_ref[...]


def _tc_copy(x):
  return pl.pallas_call(
      _copy_body,
      grid=(pl.cdiv(N, BR),),
      in_specs=[pl.BlockSpec((BR, D), lambda i: (i, 0))],
      out_specs=pl.BlockSpec((BR, D), lambda i: (i, 0)),
      out_shape=jax.ShapeDtypeStruct((N, D), jnp.float32),
  )(x)


def kernel(nodes, edge_index, W1_l, W1_r, b1, W2_l, W2_r, b2):
  src = edge_index[0]
  dst = edge_index[1]
  pad = EPAD - E
  srcp = jnp.concatenate([src, jnp.zeros((pad,), jnp.int32)]).reshape(
      NW * NCHUNK, CH)
  dstp = jnp.concatenate([dst, jnp.full((pad,), NOUT - 1, jnp.int32)]).reshape(
      NW * NCHUNK, CH)
  zrows = jnp.zeros((RPT, D), jnp.float32)
  zcnt = jnp.zeros((NOUT,), jnp.float32)

  cnts = _sc_cnt(dstp, zcnt)
  cnt_t = jnp.transpose(cnts)
  x0 = _tc_copy(nodes)
  agg1 = _sc_agg(x0, srcp, dstp, zrows)
  h1 = _tc_layer(agg1, cnt_t, x0, W1_l, W1_r, b1[None, :])
  agg2 = _sc_agg(h1, srcp, dstp, zrows)
  h2 = _tc_layer(agg2, cnt_t, h1, W2_l, W2_r, b2[None, :])
  return h2


# speedup vs baseline: 1.0899x; 1.0899x over previous
import functools

import jax
import jax.numpy as jnp
from jax import lax
from jax.experimental import pallas as pl
from jax.experimental.pallas import tpu as pltpu
from jax.experimental.pallas import tpu_sc as plsc

N = 10000
D = 128
E = 320000
NC = 2
NS = 16
NW = NC * NS
CH = 128
NCHUNK = ((-(-E // (NW * CH)) + 7) // 8) * 8
EPT = NCHUNK * CH
EPAD = EPT * NW
GB = 8
NG = NCHUNK // GB
K0 = 128
K1 = 2 * NCHUNK - K0
NOUT = 10240
RPT = NOUT // NS
BR = 512

_mesh = plsc.VectorSubcoreMesh(core_axis_name="c", subcore_axis_name="s")
_sc_params = pltpu.CompilerParams(needs_layout_passes=False)


@functools.partial(
    pl.kernel,
    mesh=_mesh,
    out_type=jax.ShapeDtypeStruct((NC, NOUT, D), jnp.float32),
    scratch_types=[
        pltpu.VMEM((GB, CH), jnp.int32),
        pltpu.VMEM((GB, CH), jnp.int32),
        pltpu.VMEM((GB, CH), jnp.int32),
        pltpu.VMEM((GB, CH), jnp.int32),
        pltpu.VMEM((2 * CH, D), jnp.float32),
        pltpu.VMEM_SHARED((NOUT, D), jnp.float32),
        pltpu.SemaphoreType.DMA,
        pltpu.SemaphoreType.DMA,
        pltpu.SemaphoreType.DMA,
        pltpu.SemaphoreType.DMA,
        pltpu.SemaphoreType.DMA,
    ],
)
def _sc_agg(x, srcp, dstp, zrows, out,
            srcA, dstA, srcB, dstB, rows2, agg_sh, g0, g1, s0, s1, isem):
  c = lax.axis_index("c")
  s = lax.axis_index("s")
  r0 = s * RPT

  gsem = (g0, g1)
  ssem = (s0, s1)
  bufs = ((srcA, dstA), (srcB, dstB))

  def rslot(u):
    return rows2.at[pl.ds(u * CH, CH)]

  def gather_start(sbuf, j, u):
    pltpu.async_copy(x.at[sbuf.at[j]], rslot(u), gsem[u])

  def gather_wait(sbuf, j, u):
    pltpu.make_async_copy(x.at[sbuf.at[j]], rslot(u), gsem[u]).wait()

  def scatter_start(dbuf, j, u):
    pltpu.async_copy(rslot(u), agg_sh.at[dbuf.at[j]], ssem[u], add=True)

  def scatter_wait(dbuf, j, u):
    pltpu.make_async_copy(rslot(u), agg_sh.at[dbuf.at[j]], ssem[u]).wait()

  def pipe(base, ngk):
    def idx_start(m, par):
      sbuf, dbuf = bufs[par]
      pltpu.async_copy(srcp.at[pl.ds(base + m * GB, GB)], sbuf, isem)
      pltpu.async_copy(dstp.at[pl.ds(base + m * GB, GB)], dbuf, isem)

    def idx_wait(m, par):
      sbuf, dbuf = bufs[par]
      pltpu.make_async_copy(srcp.at[pl.ds(base + m * GB, GB)], sbuf, isem).wait()
      pltpu.make_async_copy(dstp.at[pl.ds(base + m * GB, GB)], dbuf, isem).wait()

    def emit_group(m, par, first=False, last=False):
      sbuf, dbuf = bufs[par]
      psbuf, pdbuf = bufs[1 - par]
      for u in range(GB):
        tp = u % 2
        if u == 0:
          if not first:
            scatter_wait(pdbuf, GB - 1, 1)
          if not last:
            idx_start(m + 1, 1 - par)
        else:
          scatter_wait(dbuf, u - 1, (u - 1) % 2)
        if u < GB - 1:
          gather_start(sbuf, u + 1, (u + 1) % 2)
        elif not last:
          idx_wait(m + 1, 1 - par)
          gather_start(psbuf, 0, 0)
        gather_wait(sbuf, u, tp)
        scatter_start(dbuf, u, tp)

    pltpu.sync_copy(srcp.at[pl.ds(base, GB)], srcA)
    pltpu.sync_copy(dstp.at[pl.ds(base, GB)], dstA)
    gather_start(srcA, 0, 0)
    emit_group(0, 0, first=True)
    emit_group(1, 1)

    def pair(k, carry):
      emit_group(2 * k, 0)
      emit_group(2 * k + 1, 1)
      return carry

    lax.fori_loop(1, ngk // 2 - 1, pair, 0)

    emit_group(ngk - 2, 0)
    emit_group(ngk - 1, 1, last=True)
    scatter_wait(bufs[1][1], GB - 1, 1)

  pltpu.sync_copy(zrows, agg_sh.at[pl.ds(r0, RPT)])
  plsc.subcore_barrier()
  lax.cond(
      c == 0,
      lambda: pipe(s * K0, K0 // GB),
      lambda: pipe(NS * K0 + s * K1, K1 // GB),
  )
  plsc.subcore_barrier()
  pltpu.sync_copy(agg_sh.at[pl.ds(r0, RPT)], out.at[c, pl.ds(r0, RPT)])


@functools.partial(
    pl.kernel,
    mesh=_mesh,
    out_type=jax.ShapeDtypeStruct((NW, NOUT), jnp.float32),
    scratch_types=[
        pltpu.VMEM((NCHUNK, CH), jnp.int32),
        pltpu.VMEM((NOUT,), jnp.float32),
    ],
    compiler_params=_sc_params,
)
def _sc_cnt(dstp, zcnt, out, dst_all, cnt_v):
  c = lax.axis_index("c")
  s = lax.axis_index("s")
  w = s * NC + c
  pltpu.sync_copy(zcnt, cnt_v)
  pltpu.sync_copy(dstp.at[pl.ds(w * NCHUNK, NCHUNK)], dst_all)
  ones16 = jnp.ones((16,), jnp.float32)

  def chunk(j, carry):
    for k in range(CH // 16):
      idx = dst_all[j, pl.ds(k * 16, 16)]
      plsc.addupdate_scatter(cnt_v, [idx], ones16)
    return carry

  lax.fori_loop(0, NCHUNK, chunk, 0)
  pltpu.sync_copy(cnt_v, out.at[w])


def _tc_body(agg_ref, cnt_ref, x_ref, wl_ref, wr_ref, b_ref, out_ref):
  a = agg_ref[0] + agg_ref[1]
  cnt = jnp.sum(cnt_ref[...], axis=1, keepdims=True)
  mean = a / jnp.maximum(cnt, 1.0)
  acc = jnp.dot(mean, wl_ref[...], preferred_element_type=jnp.float32)
  acc = acc + jnp.dot(x_ref[...], wr_ref[...], preferred_element_type=jnp.float32)
  out_ref[...] = jnp.tanh(acc + b_ref[...])


def _tc_layer(agg, cnt_t, x, wl, wr, b):
  return pl.pallas_call(
      _tc_body,
      grid=(NOUT // BR,),
      in_specs=[
          pl.BlockSpec((NC, BR, D), lambda i: (0, i, 0)),
          pl.BlockSpec((BR, NW), lambda i: (i, 0)),
          pl.BlockSpec((BR, D), lambda i: (i, 0)),
          pl.BlockSpec((D, D), lambda i: (0, 0)),
          pl.BlockSpec((D, D), lambda i: (0, 0)),
          pl.BlockSpec((1, D), lambda i: (0, 0)),
      ],
      out_specs=pl.BlockSpec((BR, D), lambda i: (i, 0)),
      out_shape=jax.ShapeDtypeStruct((N, D), jnp.float32),
  )(agg, cnt_t, x, wl, wr, b)


def kernel(nodes, edge_index, W1_l, W1_r, b1, W2_l, W2_r, b2):
  src = edge_index[0]
  dst = edge_index[1]
  pad = EPAD - E
  srcp = jnp.concatenate([src, jnp.zeros((pad,), jnp.int32)]).reshape(
      NW * NCHUNK, CH)
  dstp = jnp.concatenate([dst, jnp.full((pad,), NOUT - 1, jnp.int32)]).reshape(
      NW * NCHUNK, CH)
  zrows = jnp.zeros((RPT, D), jnp.float32)
  zcnt = jnp.zeros((NOUT,), jnp.float32)

  cnts = _sc_cnt(dstp, zcnt)
  cnt_t = jnp.transpose(cnts)
  agg1 = _sc_agg(nodes, srcp, dstp, zrows)
  h1 = _tc_layer(agg1, cnt_t, nodes, W1_l, W1_r, b1[None, :])
  agg2 = _sc_agg(h1, srcp, dstp, zrows)
  h2 = _tc_layer(agg2, cnt_t, h1, W2_l, W2_r, b2[None, :])
  return h2
